# baseline (speedup 1.0000x reference)
import functools

import jax
import jax.numpy as jnp
from jax import lax
from jax.experimental import pallas as pl
from jax.experimental.pallas import tpu as pltpu
from jax.experimental.pallas import tpu_sc as plsc

N = 10000
NPAD = 10112
E = 160000
NSC = 2
NSUB = 16
EPT = E // NSUB
CHUNK = 128
NCHUNK = 80
EPT_PAD = NCHUNK * CHUNK
ROWS_PER_SUB = NPAD // NSUB
H = 192
HW = 128
HP = 256


def _sc_aggregate(hr2, ep, zeros):
    mesh = plsc.VectorSubcoreMesh(core_axis_name="c", subcore_axis_name="s")

    @functools.partial(
        pl.kernel,
        mesh=mesh,
        out_type=jax.ShapeDtypeStruct((NSC, NPAD, HW), jnp.float32),
        scratch_types=[
            pltpu.VMEM((EPT_PAD,), jnp.int32),
            pltpu.VMEM((CHUNK,), jnp.int32),
            pltpu.VMEM((CHUNK,), jnp.int32),
            pltpu.VMEM((CHUNK,), jnp.int32),
            pltpu.VMEM((CHUNK, HW), jnp.float32),
            pltpu.VMEM((CHUNK, HW), jnp.float32),
            pltpu.VMEM_SHARED((NPAD, HW), jnp.float32),
            pltpu.SemaphoreType.DMA,
        ],
    )
    def agg_kernel(hr_hbm, ep_hbm, zeros_hbm, out_hbm,
                   ep_v, sbuf0, sbuf1, dbuf, rows0, rows1, acc, sem):
        c = lax.axis_index("c")
        s = lax.axis_index("s")
        pltpu.sync_copy(zeros_hbm, acc.at[pl.ds(s * ROWS_PER_SUB, ROWS_PER_SUB)])
        pltpu.sync_copy(ep_hbm.at[c, s], ep_v)
        plsc.subcore_barrier()

        def unpack_src(j, buf):
            for k in range(CHUNK // 16):
                v = ep_v[pl.ds(j * CHUNK + k * 16, 16)]
                buf[pl.ds(k * 16, 16)] = jnp.bitwise_and(v, 0xFFFF)

        def unpack_dst(j, buf):
            for k in range(CHUNK // 16):
                v = ep_v[pl.ds(j * CHUNK + k * 16, 16)]
                buf[pl.ds(k * 16, 16)] = lax.shift_right_logical(v, 16)

        def issue(sbuf, buf):
            pltpu.async_copy(hr_hbm.at[sbuf], buf, sem)

        def wait(sbuf, buf):
            pltpu.make_async_copy(hr_hbm.at[sbuf], buf, sem).wait()

        unpack_src(0, sbuf0)
        issue(sbuf0, rows0)

        def body(i, carry):
            ja = 2 * i
            unpack_src(ja + 1, sbuf1)
            issue(sbuf1, rows1)
            wait(sbuf0, rows0)
            unpack_dst(ja, dbuf)
            pltpu.sync_copy(rows0, acc.at[dbuf], add=True)

            @pl.when(i < NCHUNK // 2 - 1)
            def _():
                unpack_src(ja + 2, sbuf0)
                issue(sbuf0, rows0)

            wait(sbuf1, rows1)
            unpack_dst(ja + 1, dbuf)
            pltpu.sync_copy(rows1, acc.at[dbuf], add=True)
            return carry

        lax.fori_loop(0, NCHUNK // 2, body, 0)
        plsc.subcore_barrier()
        pltpu.sync_copy(acc.at[pl.ds(s * ROWS_PER_SUB, ROWS_PER_SUB)],
                        out_hbm.at[c, pl.ds(s * ROWS_PER_SUB, ROWS_PER_SUB)])

    return agg_kernel(hr2, ep, zeros)


def _tc_first(x, Wr0p):
    BLK = 1000
    d = x.shape[1]

    def mm(x_ref, w_ref, o_ref):
        r = lax.dot_general(
            x_ref[...], w_ref[...], (((1,), (1,)), ((), ())),
            preferred_element_type=jnp.float32)
        o_ref[0] = r[:, :HW]
        o_ref[1] = r[:, HW:]

    return pl.pallas_call(
        mm,
        grid=(N // BLK,),
        in_specs=[pl.BlockSpec((BLK, d), lambda i: (i, 0)),
                  pl.BlockSpec(Wr0p.shape, lambda i: (0, 0))],
        out_specs=pl.BlockSpec((2, BLK, HW), lambda i: (0, i, 0)),
        out_shape=jax.ShapeDtypeStruct((2, N, HW), jnp.float32),
    )(x, Wr0p)


def _tc_layer(aggs, h, Wroot, br, Wnextp):
    BLK = 1000
    d = h.shape[1]

    def k(agg_ref, h_ref, wroot_ref, br_ref, wnext_ref, hnew_ref, hrn_ref):
        agg = jnp.concatenate([agg_ref[0], agg_ref[1][:, :H - HW]], axis=1)
        root = lax.dot_general(
            h_ref[...], wroot_ref[...], (((1,), (1,)), ((), ())),
            preferred_element_type=jnp.float32)
        hnew = jnp.maximum(agg + root + br_ref[...], 0.0)
        hnew_ref[...] = hnew
        r = lax.dot_general(
            hnew, wnext_ref[...], (((1,), (1,)), ((), ())),
            preferred_element_type=jnp.float32)
        hrn_ref[0] = r[:, :HW]
        hrn_ref[1] = r[:, HW:]

    return pl.pallas_call(
        k,
        grid=(N // BLK,),
        in_specs=[pl.BlockSpec((NSC, BLK, HW), lambda i: (0, i, 0)),
                  pl.BlockSpec((BLK, d), lambda i: (i, 0)),
                  pl.BlockSpec((H, d), lambda i: (0, 0)),
                  pl.BlockSpec((1, H), lambda i: (0, 0)),
                  pl.BlockSpec((HP, H), lambda i: (0, 0))],
        out_specs=[pl.BlockSpec((BLK, H), lambda i: (i, 0)),
                   pl.BlockSpec((2, BLK, HW), lambda i: (0, i, 0))],
        out_shape=[jax.ShapeDtypeStruct((N, H), jnp.float32),
                   jax.ShapeDtypeStruct((2, N, HW), jnp.float32)],
    )(aggs, h, Wroot, br, Wnextp)


def _tc_final(aggs, h, Wroot, br, Wlin, blin):
    BLK = 1000
    d = h.shape[1]
    DO = Wlin.shape[0]

    def k(agg_ref, h_ref, wroot_ref, br_ref, wlin_ref, blin_ref, o_ref):
        agg = jnp.concatenate([agg_ref[0], agg_ref[1][:, :H - HW]], axis=1)
        root = lax.dot_general(
            h_ref[...], wroot_ref[...], (((1,), (1,)), ((), ())),
            preferred_element_type=jnp.float32)
        hnew = jnp.maximum(agg + root + br_ref[...], 0.0)
        o_ref[...] = lax.dot_general(
            hnew, wlin_ref[...], (((1,), (1,)), ((), ())),
            preferred_element_type=jnp.float32) + blin_ref[...]

    return pl.pallas_call(
        k,
        grid=(N // BLK,),
        in_specs=[pl.BlockSpec((NSC, BLK, HW), lambda i: (0, i, 0)),
                  pl.BlockSpec((BLK, d), lambda i: (i, 0)),
                  pl.BlockSpec((H, d), lambda i: (0, 0)),
                  pl.BlockSpec((1, H), lambda i: (0, 0)),
                  pl.BlockSpec((DO, H), lambda i: (0, 0)),
                  pl.BlockSpec((1, DO), lambda i: (0, 0))],
        out_specs=pl.BlockSpec((BLK, DO), lambda i: (i, 0)),
        out_shape=jax.ShapeDtypeStruct((N, DO), jnp.float32),
    )(aggs, h, Wroot, br, Wlin, blin)


def _pad_w(Wr):
    return jnp.pad(Wr, ((0, HP - H), (0, 0)))


def kernel(x, edge_index, W_rel0, b_rel0, W_root0, W_rel1, b_rel1, W_root1,
           W_rel2, b_rel2, W_root2, W_lin, b_lin):
    src = edge_index[0]
    dst = edge_index[1]
    pad = EPT_PAD * NSUB - E
    src0 = jnp.pad(src, (0, pad), constant_values=0
                   ).reshape(NSUB, EPT_PAD)
    dsth = jnp.pad(dst, (0, pad), constant_values=N
                   ).reshape(NSUB, EPT_PAD) << 16
    ep = jnp.stack([src0 + dsth, src0 + N + dsth])
    zeros = jnp.zeros((ROWS_PER_SUB, HW), jnp.float32)

    def agg(hr):
        return _sc_aggregate(hr.reshape(2 * N, HW), ep, zeros)

    agg0 = agg(_tc_first(x, _pad_w(W_rel0)))
    h1, hr1 = _tc_layer(agg0, x, W_root0, b_rel0.reshape(1, -1),
                        _pad_w(W_rel1))
    agg1 = agg(hr1)
    h2, hr2 = _tc_layer(agg1, h1, W_root1, b_rel1.reshape(1, -1),
                        _pad_w(W_rel2))
    agg2 = agg(hr2)
    return _tc_final(agg2, h2, W_root2, b_rel2.reshape(1, -1),
                     W_lin, b_lin.reshape(1, -1))


# speedup vs baseline: 1.1367x; 1.1367x over previous
import functools

import jax
import jax.numpy as jnp
from jax import lax
from jax.experimental import pallas as pl
from jax.experimental.pallas import tpu as pltpu
from jax.experimental.pallas import tpu_sc as plsc

N = 10000
NPAD = 10112
E = 160000
NSC = 2
NSUB = 16
EPT = E // NSUB
CHUNK = 96
NCHUNK = 106
EPT_PAD = NCHUNK * CHUNK
ROWS_PER_SUB = NPAD // NSUB
H = 192
HW = 128
HP = 256


def _sc_aggregate(hr2, srcp, dstp, zeros):
    mesh = plsc.VectorSubcoreMesh(core_axis_name="c", subcore_axis_name="s")

    @functools.partial(
        pl.kernel,
        mesh=mesh,
        out_type=jax.ShapeDtypeStruct((NSC, NPAD, HW), jnp.float32),
        scratch_types=[
            pltpu.VMEM((EPT_PAD,), jnp.int32),
            pltpu.VMEM((NCHUNK, CHUNK), jnp.int32),
            pltpu.VMEM((CHUNK, HW), jnp.float32),
            pltpu.VMEM((CHUNK, HW), jnp.float32),
            pltpu.VMEM_SHARED((NPAD, HW), jnp.float32),
            pltpu.SemaphoreType.DMA,
        ],
    )
    def agg_kernel(hr_hbm, src_hbm, dst_hbm, zeros_hbm, out_hbm,
                   src_v, dst_v, rows0, rows1, acc, sem):
        c = lax.axis_index("c")
        s = lax.axis_index("s")
        pltpu.sync_copy(zeros_hbm, acc.at[pl.ds(s * ROWS_PER_SUB, ROWS_PER_SUB)])
        pltpu.sync_copy(src_hbm.at[c, s], src_v)
        pltpu.sync_copy(dst_hbm.at[s], dst_v)
        plsc.subcore_barrier()

        def issue(j, buf):
            pltpu.async_copy(hr_hbm.at[src_v.at[pl.ds(j * CHUNK, CHUNK)]],
                             buf, sem)

        def wait(j, buf):
            pltpu.make_async_copy(
                hr_hbm.at[src_v.at[pl.ds(j * CHUNK, CHUNK)]], buf, sem).wait()

        issue(0, rows0)

        def body(i, carry):
            ja = 2 * i
            issue(ja + 1, rows1)
            wait(ja, rows0)
            pltpu.sync_copy(rows0, acc.at[dst_v.at[ja]], add=True)

            @pl.when(i < NCHUNK // 2 - 1)
            def _():
                issue(ja + 2, rows0)

            wait(ja + 1, rows1)
            pltpu.sync_copy(rows1, acc.at[dst_v.at[ja + 1]], add=True)
            return carry

        lax.fori_loop(0, NCHUNK // 2, body, 0)
        plsc.subcore_barrier()
        pltpu.sync_copy(acc.at[pl.ds(s * ROWS_PER_SUB, ROWS_PER_SUB)],
                        out_hbm.at[c, pl.ds(s * ROWS_PER_SUB, ROWS_PER_SUB)])

    return agg_kernel(hr2, srcp, dstp, zeros)


def _tc_first(x, Wr0p):
    BLK = 1000
    d = x.shape[1]

    def mm(x_ref, w_ref, o_ref):
        r = lax.dot_general(
            x_ref[...], w_ref[...], (((1,), (1,)), ((), ())),
            preferred_element_type=jnp.float32)
        o_ref[0] = r[:, :HW]
        o_ref[1] = r[:, HW:]

    return pl.pallas_call(
        mm,
        grid=(N // BLK,),
        in_specs=[pl.BlockSpec((BLK, d), lambda i: (i, 0)),
                  pl.BlockSpec(Wr0p.shape, lambda i: (0, 0))],
        out_specs=pl.BlockSpec((2, BLK, HW), lambda i: (0, i, 0)),
        out_shape=jax.ShapeDtypeStruct((2, N, HW), jnp.float32),
    )(x, Wr0p)


def _tc_root(h, Wroot, br):
    BLK = 1000
    d = h.shape[1]

    def k(h_ref, wroot_ref, br_ref, o_ref):
        o_ref[...] = lax.dot_general(
            h_ref[...], wroot_ref[...], (((1,), (1,)), ((), ())),
            preferred_element_type=jnp.float32) + br_ref[...]

    return pl.pallas_call(
        k,
        grid=(N // BLK,),
        in_specs=[pl.BlockSpec((BLK, d), lambda i: (i, 0)),
                  pl.BlockSpec((H, d), lambda i: (0, 0)),
                  pl.BlockSpec((1, H), lambda i: (0, 0))],
        out_specs=pl.BlockSpec((BLK, H), lambda i: (i, 0)),
        out_shape=jax.ShapeDtypeStruct((N, H), jnp.float32),
    )(h, Wroot, br)


def _tc_combine(aggs, root, Wnextp):
    BLK = 1000

    def k(agg_ref, root_ref, wnext_ref, hnew_ref, hrn_ref):
        agg = jnp.concatenate([agg_ref[0], agg_ref[1][:, :H - HW]], axis=1)
        hnew = jnp.maximum(agg + root_ref[...], 0.0)
        hnew_ref[...] = hnew
        r = lax.dot_general(
            hnew, wnext_ref[...], (((1,), (1,)), ((), ())),
            preferred_element_type=jnp.float32)
        hrn_ref[0] = r[:, :HW]
        hrn_ref[1] = r[:, HW:]

    return pl.pallas_call(
        k,
        grid=(N // BLK,),
        in_specs=[pl.BlockSpec((NSC, BLK, HW), lambda i: (0, i, 0)),
                  pl.BlockSpec((BLK, H), lambda i: (i, 0)),
                  pl.BlockSpec((HP, H), lambda i: (0, 0))],
        out_specs=[pl.BlockSpec((BLK, H), lambda i: (i, 0)),
                   pl.BlockSpec((2, BLK, HW), lambda i: (0, i, 0))],
        out_shape=[jax.ShapeDtypeStruct((N, H), jnp.float32),
                   jax.ShapeDtypeStruct((2, N, HW), jnp.float32)],
    )(aggs, root, Wnextp)


def _tc_final(aggs, root, Wlin, blin):
    BLK = 1000
    DO = Wlin.shape[0]

    def k(agg_ref, root_ref, wlin_ref, blin_ref, o_ref):
        agg = jnp.concatenate([agg_ref[0], agg_ref[1][:, :H - HW]], axis=1)
        hnew = jnp.maximum(agg + root_ref[...], 0.0)
        o_ref[...] = lax.dot_general(
            hnew, wlin_ref[...], (((1,), (1,)), ((), ())),
            preferred_element_type=jnp.float32) + blin_ref[...]

    return pl.pallas_call(
        k,
        grid=(N // BLK,),
        in_specs=[pl.BlockSpec((NSC, BLK, HW), lambda i: (0, i, 0)),
                  pl.BlockSpec((BLK, H), lambda i: (i, 0)),
                  pl.BlockSpec((DO, H), lambda i: (0, 0)),
                  pl.BlockSpec((1, DO), lambda i: (0, 0))],
        out_specs=pl.BlockSpec((BLK, DO), lambda i: (i, 0)),
        out_shape=jax.ShapeDtypeStruct((N, DO), jnp.float32),
    )(aggs, root, Wlin, blin)


def _pad_w(Wr):
    return jnp.pad(Wr, ((0, HP - H), (0, 0)))


def kernel(x, edge_index, W_rel0, b_rel0, W_root0, W_rel1, b_rel1, W_root1,
           W_rel2, b_rel2, W_root2, W_lin, b_lin):
    src = edge_index[0]
    dst = edge_index[1]
    pad = EPT_PAD * NSUB - E
    src0 = jnp.pad(src, (0, pad), constant_values=0
                   ).reshape(NSUB, EPT_PAD)
    srcp = jnp.stack([src0, src0 + N])
    dstp = jnp.pad(dst, (0, pad), constant_values=N
                   ).reshape(NSUB, NCHUNK, CHUNK)
    zeros = jnp.zeros((ROWS_PER_SUB, HW), jnp.float32)

    def agg(hr):
        return _sc_aggregate(hr.reshape(2 * N, HW), srcp, dstp, zeros)

    hr0 = _tc_first(x, _pad_w(W_rel0))
    root0 = _tc_root(x, W_root0, b_rel0.reshape(1, -1))
    agg0 = agg(hr0)
    h1, hr1 = _tc_combine(agg0, root0, _pad_w(W_rel1))
    root1 = _tc_root(h1, W_root1, b_rel1.reshape(1, -1))
    agg1 = agg(hr1)
    h2, hr2 = _tc_combine(agg1, root1, _pad_w(W_rel2))
    root2 = _tc_root(h2, W_root2, b_rel2.reshape(1, -1))
    agg2 = agg(hr2)
    return _tc_final(agg2, root2, W_lin, b_lin.reshape(1, -1))


# speedup vs baseline: 1.1600x; 1.0205x over previous
import functools

import jax
import jax.numpy as jnp
from jax import lax
from jax.experimental import pallas as pl
from jax.experimental.pallas import tpu as pltpu
from jax.experimental.pallas import tpu_sc as plsc

N = 10000
NPAD = 10112
E = 160000
NSC = 2
NSUB = 16
EPT = E // NSUB
CHUNK = 104
NCHUNK = 98
EPT_PAD = NCHUNK * CHUNK
ROWS_PER_SUB = NPAD // NSUB
H = 192
HW = 128
HP = 256


def _sc_aggregate(hr2, srcp, dstp, zeros):
    mesh = plsc.VectorSubcoreMesh(core_axis_name="c", subcore_axis_name="s")

    @functools.partial(
        pl.kernel,
        mesh=mesh,
        out_type=jax.ShapeDtypeStruct((NSC, NPAD, HW), jnp.float32),
        scratch_types=[
            pltpu.VMEM((EPT_PAD,), jnp.int32),
            pltpu.VMEM((NCHUNK, CHUNK), jnp.int32),
            pltpu.VMEM((CHUNK, HW), jnp.float32),
            pltpu.VMEM((CHUNK, HW), jnp.float32),
            pltpu.VMEM_SHARED((NPAD, HW), jnp.float32),
            pltpu.SemaphoreType.DMA,
        ],
    )
    def agg_kernel(hr_hbm, src_hbm, dst_hbm, zeros_hbm, out_hbm,
                   src_v, dst_v, rows0, rows1, acc, sem):
        c = lax.axis_index("c")
        s = lax.axis_index("s")
        pltpu.sync_copy(zeros_hbm, acc.at[pl.ds(s * ROWS_PER_SUB, ROWS_PER_SUB)])
        pltpu.sync_copy(src_hbm.at[c, s], src_v)
        pltpu.sync_copy(dst_hbm.at[s], dst_v)
        plsc.subcore_barrier()

        def issue(j, buf):
            pltpu.async_copy(hr_hbm.at[src_v.at[pl.ds(j * CHUNK, CHUNK)]],
                             buf, sem)

        def wait(j, buf):
            pltpu.make_async_copy(
                hr_hbm.at[src_v.at[pl.ds(j * CHUNK, CHUNK)]], buf, sem).wait()

        issue(0, rows0)

        def body(i, carry):
            ja = 2 * i
            issue(ja + 1, rows1)
            wait(ja, rows0)
            pltpu.sync_copy(rows0, acc.at[dst_v.at[ja]], add=True)

            @pl.when(i < NCHUNK // 2 - 1)
            def _():
                issue(ja + 2, rows0)

            wait(ja + 1, rows1)
            pltpu.sync_copy(rows1, acc.at[dst_v.at[ja + 1]], add=True)
            return carry

        lax.fori_loop(0, NCHUNK // 2, body, 0)
        plsc.subcore_barrier()
        pltpu.sync_copy(acc.at[pl.ds(s * ROWS_PER_SUB, ROWS_PER_SUB)],
                        out_hbm.at[c, pl.ds(s * ROWS_PER_SUB, ROWS_PER_SUB)])

    return agg_kernel(hr2, srcp, dstp, zeros)


def _tc_first(x, Wr0p):
    BLK = 1000
    d = x.shape[1]

    def mm(x_ref, w_ref, o_ref):
        r = lax.dot_general(
            x_ref[...], w_ref[...], (((1,), (1,)), ((), ())),
            preferred_element_type=jnp.float32)
        o_ref[0] = r[:, :HW]
        o_ref[1] = r[:, HW:]

    return pl.pallas_call(
        mm,
        grid=(N // BLK,),
        in_specs=[pl.BlockSpec((BLK, d), lambda i: (i, 0)),
                  pl.BlockSpec(Wr0p.shape, lambda i: (0, 0))],
        out_specs=pl.BlockSpec((2, BLK, HW), lambda i: (0, i, 0)),
        out_shape=jax.ShapeDtypeStruct((2, N, HW), jnp.float32),
    )(x, Wr0p)


def _tc_layer(aggs, h, Wroot, br, Wnextp):
    BLK = 1000
    d = h.shape[1]

    def k(agg_ref, h_ref, wroot_ref, br_ref, wnext_ref, hnew_ref, hrn_ref):
        agg = jnp.concatenate([agg_ref[0], agg_ref[1][:, :H - HW]], axis=1)
        root = lax.dot_general(
            h_ref[...], wroot_ref[...], (((1,), (1,)), ((), ())),
            preferred_element_type=jnp.float32)
        hnew = jnp.maximum(agg + root + br_ref[...], 0.0)
        hnew_ref[...] = hnew
        r = lax.dot_general(
            hnew, wnext_ref[...], (((1,), (1,)), ((), ())),
            preferred_element_type=jnp.float32)
        hrn_ref[0] = r[:, :HW]
        hrn_ref[1] = r[:, HW:]

    return pl.pallas_call(
        k,
        grid=(N // BLK,),
        in_specs=[pl.BlockSpec((NSC, BLK, HW), lambda i: (0, i, 0)),
                  pl.BlockSpec((BLK, d), lambda i: (i, 0)),
                  pl.BlockSpec((H, d), lambda i: (0, 0)),
                  pl.BlockSpec((1, H), lambda i: (0, 0)),
                  pl.BlockSpec((HP, H), lambda i: (0, 0))],
        out_specs=[pl.BlockSpec((BLK, H), lambda i: (i, 0)),
                   pl.BlockSpec((2, BLK, HW), lambda i: (0, i, 0))],
        out_shape=[jax.ShapeDtypeStruct((N, H), jnp.float32),
                   jax.ShapeDtypeStruct((2, N, HW), jnp.float32)],
    )(aggs, h, Wroot, br, Wnextp)


def _tc_final(aggs, h, Wroot, br, Wlin, blin):
    BLK = 1000
    d = h.shape[1]
    DO = Wlin.shape[0]

    def k(agg_ref, h_ref, wroot_ref, br_ref, wlin_ref, blin_ref, o_ref):
        agg = jnp.concatenate([agg_ref[0], agg_ref[1][:, :H - HW]], axis=1)
        root = lax.dot_general(
            h_ref[...], wroot_ref[...], (((1,), (1,)), ((), ())),
            preferred_element_type=jnp.float32)
        hnew = jnp.maximum(agg + root + br_ref[...], 0.0)
        o_ref[...] = lax.dot_general(
            hnew, wlin_ref[...], (((1,), (1,)), ((), ())),
            preferred_element_type=jnp.float32) + blin_ref[...]

    return pl.pallas_call(
        k,
        grid=(N // BLK,),
        in_specs=[pl.BlockSpec((NSC, BLK, HW), lambda i: (0, i, 0)),
                  pl.BlockSpec((BLK, d), lambda i: (i, 0)),
                  pl.BlockSpec((H, d), lambda i: (0, 0)),
                  pl.BlockSpec((1, H), lambda i: (0, 0)),
                  pl.BlockSpec((DO, H), lambda i: (0, 0)),
                  pl.BlockSpec((1, DO), lambda i: (0, 0))],
        out_specs=pl.BlockSpec((BLK, DO), lambda i: (i, 0)),
        out_shape=jax.ShapeDtypeStruct((N, DO), jnp.float32),
    )(aggs, h, Wroot, br, Wlin, blin)


def _pad_w(Wr):
    return jnp.pad(Wr, ((0, HP - H), (0, 0)))


def kernel(x, edge_index, W_rel0, b_rel0, W_root0, W_rel1, b_rel1, W_root1,
           W_rel2, b_rel2, W_root2, W_lin, b_lin):
    src = edge_index[0]
    dst = edge_index[1]
    pad = EPT_PAD * NSUB - E
    src0 = jnp.pad(src, (0, pad), constant_values=0
                   ).reshape(NSUB, EPT_PAD)
    srcp = jnp.stack([src0, src0 + N])
    dstp = jnp.pad(dst, (0, pad), constant_values=N
                   ).reshape(NSUB, NCHUNK, CHUNK)
    zeros = jnp.zeros((ROWS_PER_SUB, HW), jnp.float32)

    def agg(hr):
        return _sc_aggregate(hr.reshape(2 * N, HW), srcp, dstp, zeros)

    agg0 = agg(_tc_first(x, _pad_w(W_rel0)))
    h1, hr1 = _tc_layer(agg0, x, W_root0, b_rel0.reshape(1, -1),
                        _pad_w(W_rel1))
    agg1 = agg(hr1)
    h2, hr2 = _tc_layer(agg1, h1, W_root1, b_rel1.reshape(1, -1),
                        _pad_w(W_rel2))
    agg2 = agg(hr2)
    return _tc_final(agg2, h2, W_root2, b_rel2.reshape(1, -1),
                     W_lin, b_lin.reshape(1, -1))


# speedup vs baseline: 1.1672x; 1.0062x over previous
import functools

import jax
import jax.numpy as jnp
from jax import lax
from jax.experimental import pallas as pl
from jax.experimental.pallas import tpu as pltpu
from jax.experimental.pallas import tpu_sc as plsc

N = 10000
NPAD = 10112
E = 160000
NSC = 2
NSUB = 16
EPT = E // NSUB
CHUNK = 96
NCHUNK = 106
EPT_PAD = NCHUNK * CHUNK
ROWS_PER_SUB = NPAD // NSUB
H = 192
HW = 128
HP = 256


def _sc_aggregate(hr2, srcp, dstp, zeros):
    mesh = plsc.VectorSubcoreMesh(core_axis_name="c", subcore_axis_name="s")

    @functools.partial(
        pl.kernel,
        mesh=mesh,
        out_type=jax.ShapeDtypeStruct((NSC, NPAD, HW), jnp.float32),
        scratch_types=[
            pltpu.VMEM((EPT_PAD,), jnp.int32),
            pltpu.VMEM((NCHUNK, CHUNK), jnp.int32),
            pltpu.VMEM((CHUNK, HW), jnp.float32),
            pltpu.VMEM((CHUNK, HW), jnp.float32),
            pltpu.VMEM_SHARED((NPAD, HW), jnp.float32),
            pltpu.SemaphoreType.DMA,
        ],
    )
    def agg_kernel(hr_hbm, src_hbm, dst_hbm, zeros_hbm, out_hbm,
                   src_v, dst_v, rows0, rows1, acc, sem):
        c = lax.axis_index("c")
        s = lax.axis_index("s")
        pltpu.sync_copy(zeros_hbm, acc.at[pl.ds(s * ROWS_PER_SUB, ROWS_PER_SUB)])
        pltpu.sync_copy(src_hbm.at[c, s], src_v)
        pltpu.sync_copy(dst_hbm.at[s], dst_v)
        plsc.subcore_barrier()

        def issue(j, buf):
            pltpu.async_copy(hr_hbm.at[src_v.at[pl.ds(j * CHUNK, CHUNK)]],
                             buf, sem)

        def wait(j, buf):
            pltpu.make_async_copy(
                hr_hbm.at[src_v.at[pl.ds(j * CHUNK, CHUNK)]], buf, sem).wait()

        issue(0, rows0)

        def body(i, carry):
            ja = 2 * i
            issue(ja + 1, rows1)
            wait(ja, rows0)
            pltpu.sync_copy(rows0, acc.at[dst_v.at[ja]], add=True)

            @pl.when(i < NCHUNK // 2 - 1)
            def _():
                issue(ja + 2, rows0)

            wait(ja + 1, rows1)
            pltpu.sync_copy(rows1, acc.at[dst_v.at[ja + 1]], add=True)
            return carry

        lax.fori_loop(0, NCHUNK // 2, body, 0)
        plsc.subcore_barrier()
        pltpu.sync_copy(acc.at[pl.ds(s * ROWS_PER_SUB, ROWS_PER_SUB)],
                        out_hbm.at[c, pl.ds(s * ROWS_PER_SUB, ROWS_PER_SUB)])

    return agg_kernel(hr2, srcp, dstp, zeros)


def _tc_layer0(aggx, x, Wr0, Wroot, br, Wnextp):
    BLK = 1000
    d = x.shape[1]

    def k(agg_ref, x_ref, wr0_ref, wroot_ref, br_ref, wnext_ref,
          hnew_ref, hrn_ref):
        aggx = jnp.concatenate([agg_ref[0], agg_ref[1]], axis=1)
        rel = lax.dot_general(
            aggx, wr0_ref[...], (((1,), (1,)), ((), ())),
            preferred_element_type=jnp.float32)
        root = lax.dot_general(
            x_ref[...], wroot_ref[...], (((1,), (1,)), ((), ())),
            preferred_element_type=jnp.float32)
        hnew = jnp.maximum(rel + root + br_ref[...], 0.0)
        hnew_ref[...] = hnew
        r = lax.dot_general(
            hnew, wnext_ref[...], (((1,), (1,)), ((), ())),
            preferred_element_type=jnp.float32)
        hrn_ref[0] = r[:, :HW]
        hrn_ref[1] = r[:, HW:]

    return pl.pallas_call(
        k,
        grid=(N // BLK,),
        in_specs=[pl.BlockSpec((NSC, BLK, HW), lambda i: (0, i, 0)),
                  pl.BlockSpec((BLK, d), lambda i: (i, 0)),
                  pl.BlockSpec((H, d), lambda i: (0, 0)),
                  pl.BlockSpec((H, d), lambda i: (0, 0)),
                  pl.BlockSpec((1, H), lambda i: (0, 0)),
                  pl.BlockSpec((HP, H), lambda i: (0, 0))],
        out_specs=[pl.BlockSpec((BLK, H), lambda i: (i, 0)),
                   pl.BlockSpec((2, BLK, HW), lambda i: (0, i, 0))],
        out_shape=[jax.ShapeDtypeStruct((N, H), jnp.float32),
                   jax.ShapeDtypeStruct((2, N, HW), jnp.float32)],
    )(aggx, x, Wr0, Wroot, br, Wnextp)


def _tc_layer(aggs, h, Wroot, br, Wnextp):
    BLK = 1000
    d = h.shape[1]

    def k(agg_ref, h_ref, wroot_ref, br_ref, wnext_ref, hnew_ref, hrn_ref):
        agg = jnp.concatenate([agg_ref[0], agg_ref[1][:, :H - HW]], axis=1)
        root = lax.dot_general(
            h_ref[...], wroot_ref[...], (((1,), (1,)), ((), ())),
            preferred_element_type=jnp.float32)
        hnew = jnp.maximum(agg + root + br_ref[...], 0.0)
        hnew_ref[...] = hnew
        r = lax.dot_general(
            hnew, wnext_ref[...], (((1,), (1,)), ((), ())),
            preferred_element_type=jnp.float32)
        hrn_ref[0] = r[:, :HW]
        hrn_ref[1] = r[:, HW:]

    return pl.pallas_call(
        k,
        grid=(N // BLK,),
        in_specs=[pl.BlockSpec((NSC, BLK, HW), lambda i: (0, i, 0)),
                  pl.BlockSpec((BLK, d), lambda i: (i, 0)),
                  pl.BlockSpec((H, d), lambda i: (0, 0)),
                  pl.BlockSpec((1, H), lambda i: (0, 0)),
                  pl.BlockSpec((HP, H), lambda i: (0, 0))],
        out_specs=[pl.BlockSpec((BLK, H), lambda i: (i, 0)),
                   pl.BlockSpec((2, BLK, HW), lambda i: (0, i, 0))],
        out_shape=[jax.ShapeDtypeStruct((N, H), jnp.float32),
                   jax.ShapeDtypeStruct((2, N, HW), jnp.float32)],
    )(aggs, h, Wroot, br, Wnextp)


def _tc_final(aggs, h, Wroot, br, Wlin, blin):
    BLK = 1000
    d = h.shape[1]
    DO = Wlin.shape[0]

    def k(agg_ref, h_ref, wroot_ref, br_ref, wlin_ref, blin_ref, o_ref):
        agg = jnp.concatenate([agg_ref[0], agg_ref[1][:, :H - HW]], axis=1)
        root = lax.dot_general(
            h_ref[...], wroot_ref[...], (((1,), (1,)), ((), ())),
            preferred_element_type=jnp.float32)
        hnew = jnp.maximum(agg + root + br_ref[...], 0.0)
        o_ref[...] = lax.dot_general(
            hnew, wlin_ref[...], (((1,), (1,)), ((), ())),
            preferred_element_type=jnp.float32) + blin_ref[...]

    return pl.pallas_call(
        k,
        grid=(N // BLK,),
        in_specs=[pl.BlockSpec((NSC, BLK, HW), lambda i: (0, i, 0)),
                  pl.BlockSpec((BLK, d), lambda i: (i, 0)),
                  pl.BlockSpec((H, d), lambda i: (0, 0)),
                  pl.BlockSpec((1, H), lambda i: (0, 0)),
                  pl.BlockSpec((DO, H), lambda i: (0, 0)),
                  pl.BlockSpec((1, DO), lambda i: (0, 0))],
        out_specs=pl.BlockSpec((BLK, DO), lambda i: (i, 0)),
        out_shape=jax.ShapeDtypeStruct((N, DO), jnp.float32),
    )(aggs, h, Wroot, br, Wlin, blin)


def _pad_w(Wr):
    return jnp.pad(Wr, ((0, HP - H), (0, 0)))


def kernel(x, edge_index, W_rel0, b_rel0, W_root0, W_rel1, b_rel1, W_root1,
           W_rel2, b_rel2, W_root2, W_lin, b_lin):
    src = edge_index[0]
    dst = edge_index[1]
    pad = EPT_PAD * NSUB - E
    src0 = jnp.pad(src, (0, pad), constant_values=0
                   ).reshape(NSUB, EPT_PAD)
    srcp = jnp.stack([src0, src0 + N])
    srcx = jnp.stack([2 * src0, 2 * src0 + 1])
    dstp = jnp.pad(dst, (0, pad), constant_values=N
                   ).reshape(NSUB, NCHUNK, CHUNK)
    zeros = jnp.zeros((ROWS_PER_SUB, HW), jnp.float32)

    def agg(hr):
        return _sc_aggregate(hr.reshape(2 * N, HW), srcp, dstp, zeros)

    agg0 = _sc_aggregate(x.reshape(2 * N, HW), srcx, dstp, zeros)
    h1, hr1 = _tc_layer0(agg0, x, W_rel0, W_root0, b_rel0.reshape(1, -1),
                         _pad_w(W_rel1))
    agg1 = agg(hr1)
    h2, hr2 = _tc_layer(agg1, h1, W_root1, b_rel1.reshape(1, -1),
                        _pad_w(W_rel2))
    agg2 = agg(hr2)
    return _tc_final(agg2, h2, W_root2, b_rel2.reshape(1, -1),
                     W_lin, b_lin.reshape(1, -1))


# speedup vs baseline: 1.1988x; 1.0271x over previous
import functools

import jax
import jax.numpy as jnp
from jax import lax
from jax.experimental import pallas as pl
from jax.experimental.pallas import tpu as pltpu
from jax.experimental.pallas import tpu_sc as plsc

N = 10000
NPAD = 10112
E = 160000
NSC = 2
NSUB = 16
EPT = E // NSUB
CHUNK = 96
NCHUNK = 106
EPT_PAD = NCHUNK * CHUNK
ROWS_PER_SUB = NPAD // NSUB
H = 192
HW = 128
HP = 256


def _sc_aggregate(hr2, srcp, dstp, zeros):
    mesh = plsc.VectorSubcoreMesh(core_axis_name="c", subcore_axis_name="s")

    @functools.partial(
        pl.kernel,
        mesh=mesh,
        out_type=jax.ShapeDtypeStruct((NSC, NPAD, HW), jnp.float32),
        scratch_types=[
            pltpu.VMEM((EPT_PAD,), jnp.int32),
            pltpu.VMEM((NCHUNK, CHUNK), jnp.int32),
            pltpu.VMEM((CHUNK, HW), jnp.float32),
            pltpu.VMEM((CHUNK, HW), jnp.float32),
            pltpu.VMEM_SHARED((NPAD, HW), jnp.float32),
            pltpu.SemaphoreType.DMA,
        ],
    )
    def agg_kernel(hr_hbm, src_hbm, dst_hbm, zeros_hbm, out_hbm,
                   src_v, dst_v, rows0, rows1, acc, sem):
        c = lax.axis_index("c")
        s = lax.axis_index("s")
        pltpu.sync_copy(zeros_hbm, acc.at[pl.ds(s * ROWS_PER_SUB, ROWS_PER_SUB)])
        pltpu.sync_copy(src_hbm.at[s], src_v)
        pltpu.sync_copy(dst_hbm.at[s], dst_v)
        plsc.subcore_barrier()

        def issue(j, buf):
            idx = src_v.at[pl.ds(j * CHUNK, CHUNK)]

            @pl.when(c == 0)
            def _():
                pltpu.async_copy(hr_hbm.at[idx, pl.ds(0, HW)], buf, sem)

            @pl.when(c == 1)
            def _():
                pltpu.async_copy(hr_hbm.at[idx, pl.ds(HW, HW)], buf, sem)

        def wait(j, buf):
            pltpu.make_async_copy(
                hr_hbm.at[src_v.at[pl.ds(j * CHUNK, CHUNK)], pl.ds(0, HW)],
                buf, sem).wait()

        issue(0, rows0)

        def body(i, carry):
            ja = 2 * i
            issue(ja + 1, rows1)
            wait(ja, rows0)
            pltpu.sync_copy(rows0, acc.at[dst_v.at[ja]], add=True)

            @pl.when(i < NCHUNK // 2 - 1)
            def _():
                issue(ja + 2, rows0)

            wait(ja + 1, rows1)
            pltpu.sync_copy(rows1, acc.at[dst_v.at[ja + 1]], add=True)
            return carry

        lax.fori_loop(0, NCHUNK // 2, body, 0)
        plsc.subcore_barrier()
        pltpu.sync_copy(acc.at[pl.ds(s * ROWS_PER_SUB, ROWS_PER_SUB)],
                        out_hbm.at[c, pl.ds(s * ROWS_PER_SUB, ROWS_PER_SUB)])

    return agg_kernel(hr2, srcp, dstp, zeros)


def _tc_layer0(aggx, x, Wr0, Wroot, br, Wnextp):
    BLK = 1000
    d = x.shape[1]

    def k(agg_ref, x_ref, wr0_ref, wroot_ref, br_ref, wnext_ref,
          hnew_ref, hrn_ref):
        aggx = jnp.concatenate([agg_ref[0], agg_ref[1]], axis=1)
        rel = lax.dot_general(
            aggx, wr0_ref[...], (((1,), (1,)), ((), ())),
            preferred_element_type=jnp.float32)
        root = lax.dot_general(
            x_ref[...], wroot_ref[...], (((1,), (1,)), ((), ())),
            preferred_element_type=jnp.float32)
        hnew = jnp.maximum(rel + root + br_ref[...], 0.0)
        hnew_ref[...] = hnew
        hrn_ref[...] = lax.dot_general(
            hnew, wnext_ref[...], (((1,), (1,)), ((), ())),
            preferred_element_type=jnp.float32)

    return pl.pallas_call(
        k,
        grid=(N // BLK,),
        in_specs=[pl.BlockSpec((NSC, BLK, HW), lambda i: (0, i, 0)),
                  pl.BlockSpec((BLK, d), lambda i: (i, 0)),
                  pl.BlockSpec((H, d), lambda i: (0, 0)),
                  pl.BlockSpec((H, d), lambda i: (0, 0)),
                  pl.BlockSpec((1, H), lambda i: (0, 0)),
                  pl.BlockSpec((HP, H), lambda i: (0, 0))],
        out_specs=[pl.BlockSpec((BLK, H), lambda i: (i, 0)),
                   pl.BlockSpec((BLK, HP), lambda i: (i, 0))],
        out_shape=[jax.ShapeDtypeStruct((N, H), jnp.float32),
                   jax.ShapeDtypeStruct((N, HP), jnp.float32)],
    )(aggx, x, Wr0, Wroot, br, Wnextp)


def _tc_layer(aggs, h, Wroot, br, Wnextp):
    BLK = 1000
    d = h.shape[1]

    def k(agg_ref, h_ref, wroot_ref, br_ref, wnext_ref, hnew_ref, hrn_ref):
        agg = jnp.concatenate([agg_ref[0], agg_ref[1][:, :H - HW]], axis=1)
        root = lax.dot_general(
            h_ref[...], wroot_ref[...], (((1,), (1,)), ((), ())),
            preferred_element_type=jnp.float32)
        hnew = jnp.maximum(agg + root + br_ref[...], 0.0)
        hnew_ref[...] = hnew
        hrn_ref[...] = lax.dot_general(
            hnew, wnext_ref[...], (((1,), (1,)), ((), ())),
            preferred_element_type=jnp.float32)

    return pl.pallas_call(
        k,
        grid=(N // BLK,),
        in_specs=[pl.BlockSpec((NSC, BLK, HW), lambda i: (0, i, 0)),
                  pl.BlockSpec((BLK, d), lambda i: (i, 0)),
                  pl.BlockSpec((H, d), lambda i: (0, 0)),
                  pl.BlockSpec((1, H), lambda i: (0, 0)),
                  pl.BlockSpec((HP, H), lambda i: (0, 0))],
        out_specs=[pl.BlockSpec((BLK, H), lambda i: (i, 0)),
                   pl.BlockSpec((BLK, HP), lambda i: (i, 0))],
        out_shape=[jax.ShapeDtypeStruct((N, H), jnp.float32),
                   jax.ShapeDtypeStruct((N, HP), jnp.float32)],
    )(aggs, h, Wroot, br, Wnextp)


def _tc_final(aggs, h, Wroot, br, Wlin, blin):
    BLK = 1000
    d = h.shape[1]
    DO = Wlin.shape[0]

    def k(agg_ref, h_ref, wroot_ref, br_ref, wlin_ref, blin_ref, o_ref):
        agg = jnp.concatenate([agg_ref[0], agg_ref[1][:, :H - HW]], axis=1)
        root = lax.dot_general(
            h_ref[...], wroot_ref[...], (((1,), (1,)), ((), ())),
            preferred_element_type=jnp.float32)
        hnew = jnp.maximum(agg + root + br_ref[...], 0.0)
        o_ref[...] = lax.dot_general(
            hnew, wlin_ref[...], (((1,), (1,)), ((), ())),
            preferred_element_type=jnp.float32) + blin_ref[...]

    return pl.pallas_call(
        k,
        grid=(N // BLK,),
        in_specs=[pl.BlockSpec((NSC, BLK, HW), lambda i: (0, i, 0)),
                  pl.BlockSpec((BLK, d), lambda i: (i, 0)),
                  pl.BlockSpec((H, d), lambda i: (0, 0)),
                  pl.BlockSpec((1, H), lambda i: (0, 0)),
                  pl.BlockSpec((DO, H), lambda i: (0, 0)),
                  pl.BlockSpec((1, DO), lambda i: (0, 0))],
        out_specs=pl.BlockSpec((BLK, DO), lambda i: (i, 0)),
        out_shape=jax.ShapeDtypeStruct((N, DO), jnp.float32),
    )(aggs, h, Wroot, br, Wlin, blin)


def _pad_w(Wr):
    return jnp.pad(Wr, ((0, HP - H), (0, 0)))


def kernel(x, edge_index, W_rel0, b_rel0, W_root0, W_rel1, b_rel1, W_root1,
           W_rel2, b_rel2, W_root2, W_lin, b_lin):
    src = edge_index[0]
    dst = edge_index[1]
    pad = EPT_PAD * NSUB - E
    srcp = jnp.pad(src, (0, pad), constant_values=0
                   ).reshape(NSUB, EPT_PAD)
    dstp = jnp.pad(dst, (0, pad), constant_values=N
                   ).reshape(NSUB, NCHUNK, CHUNK)
    zeros = jnp.zeros((ROWS_PER_SUB, HW), jnp.float32)

    def agg(hr):
        return _sc_aggregate(hr, srcp, dstp, zeros)

    agg0 = agg(x)
    h1, hr1 = _tc_layer0(agg0, x, W_rel0, W_root0, b_rel0.reshape(1, -1),
                         _pad_w(W_rel1))
    agg1 = agg(hr1)
    h2, hr2 = _tc_layer(agg1, h1, W_root1, b_rel1.reshape(1, -1),
                        _pad_w(W_rel2))
    agg2 = agg(hr2)
    return _tc_final(agg2, h2, W_root2, b_rel2.reshape(1, -1),
                     W_lin, b_lin.reshape(1, -1))


# speedup vs baseline: 1.2823x; 1.0697x over previous
import functools

import jax
import jax.numpy as jnp
from jax import lax
from jax.experimental import pallas as pl
from jax.experimental.pallas import tpu as pltpu
from jax.experimental.pallas import tpu_sc as plsc

N = 10000
NPAD = 10112
E = 160000
NSC = 2
NSUB = 16
EPT = E // NSUB
CHUNK = 96
NCHUNK = 106
EPT_PAD = NCHUNK * CHUNK
ROWS_PER_SUB = NPAD // NSUB
H = 192
HW = 128
HP = 256


def _sc_aggregate(hr2, srcp, dstp, zeros, col_split):
    mesh = plsc.VectorSubcoreMesh(core_axis_name="c", subcore_axis_name="s")

    @functools.partial(
        pl.kernel,
        mesh=mesh,
        out_type=jax.ShapeDtypeStruct((NSC, NPAD, HW), jnp.float32),
        scratch_types=[
            pltpu.VMEM((EPT_PAD,), jnp.int32),
            pltpu.VMEM((NCHUNK, CHUNK), jnp.int32),
            pltpu.VMEM((CHUNK, HW), jnp.float32),
            pltpu.VMEM((CHUNK, HW), jnp.float32),
            pltpu.VMEM_SHARED((NPAD, HW), jnp.float32),
            pltpu.SemaphoreType.DMA,
        ],
    )
    def agg_kernel(hr_hbm, src_hbm, dst_hbm, zeros_hbm, out_hbm,
                   src_v, dst_v, rows0, rows1, acc, sem):
        c = lax.axis_index("c")
        s = lax.axis_index("s")
        pltpu.sync_copy(zeros_hbm, acc.at[pl.ds(s * ROWS_PER_SUB, ROWS_PER_SUB)])
        if col_split:
            pltpu.sync_copy(src_hbm.at[s], src_v)
        else:
            pltpu.sync_copy(src_hbm.at[c, s], src_v)
        pltpu.sync_copy(dst_hbm.at[s], dst_v)
        plsc.subcore_barrier()

        def issue(j, buf):
            idx = src_v.at[pl.ds(j * CHUNK, CHUNK)]
            if not col_split:
                pltpu.async_copy(hr_hbm.at[idx], buf, sem)
                return

            @pl.when(c == 0)
            def _():
                pltpu.async_copy(hr_hbm.at[idx, pl.ds(0, HW)], buf, sem)

            @pl.when(c == 1)
            def _():
                pltpu.async_copy(hr_hbm.at[idx, pl.ds(HW, HW)], buf, sem)

        def wait(j, buf):
            idx = src_v.at[pl.ds(j * CHUNK, CHUNK)]
            if col_split:
                pltpu.make_async_copy(
                    hr_hbm.at[idx, pl.ds(0, HW)], buf, sem).wait()
            else:
                pltpu.make_async_copy(hr_hbm.at[idx], buf, sem).wait()

        issue(0, rows0)

        def body(i, carry):
            ja = 2 * i
            issue(ja + 1, rows1)
            wait(ja, rows0)
            pltpu.sync_copy(rows0, acc.at[dst_v.at[ja]], add=True)

            @pl.when(i < NCHUNK // 2 - 1)
            def _():
                issue(ja + 2, rows0)

            wait(ja + 1, rows1)
            pltpu.sync_copy(rows1, acc.at[dst_v.at[ja + 1]], add=True)
            return carry

        lax.fori_loop(0, NCHUNK // 2, body, 0)
        plsc.subcore_barrier()
        pltpu.sync_copy(acc.at[pl.ds(s * ROWS_PER_SUB, ROWS_PER_SUB)],
                        out_hbm.at[c, pl.ds(s * ROWS_PER_SUB, ROWS_PER_SUB)])

    return agg_kernel(hr2, srcp, dstp, zeros)


def _tc_layer0(aggx, x, Wr0, Wroot, br, Wnextp):
    BLK = 1000
    d = x.shape[1]

    def k(agg_ref, x_ref, wr0_ref, wroot_ref, br_ref, wnext_ref,
          hnew_ref, hrn_ref):
        aggx = jnp.concatenate([agg_ref[0], agg_ref[1]], axis=1)
        rel = lax.dot_general(
            aggx, wr0_ref[...], (((1,), (1,)), ((), ())),
            preferred_element_type=jnp.float32)
        root = lax.dot_general(
            x_ref[...], wroot_ref[...], (((1,), (1,)), ((), ())),
            preferred_element_type=jnp.float32)
        hnew = jnp.maximum(rel + root + br_ref[...], 0.0)
        hnew_ref[...] = hnew
        r = lax.dot_general(
            hnew, wnext_ref[...], (((1,), (1,)), ((), ())),
            preferred_element_type=jnp.float32)
        hrn_ref[0] = r[:, :HW]
        hrn_ref[1] = r[:, HW:]

    return pl.pallas_call(
        k,
        grid=(N // BLK,),
        in_specs=[pl.BlockSpec((NSC, BLK, HW), lambda i: (0, i, 0)),
                  pl.BlockSpec((BLK, d), lambda i: (i, 0)),
                  pl.BlockSpec((H, d), lambda i: (0, 0)),
                  pl.BlockSpec((H, d), lambda i: (0, 0)),
                  pl.BlockSpec((1, H), lambda i: (0, 0)),
                  pl.BlockSpec((HP, H), lambda i: (0, 0))],
        out_specs=[pl.BlockSpec((BLK, H), lambda i: (i, 0)),
                   pl.BlockSpec((2, BLK, HW), lambda i: (0, i, 0))],
        out_shape=[jax.ShapeDtypeStruct((N, H), jnp.float32),
                   jax.ShapeDtypeStruct((2, N, HW), jnp.float32)],
    )(aggx, x, Wr0, Wroot, br, Wnextp)


def _tc_layer(aggs, h, Wroot, br, Wnextp):
    BLK = 1000
    d = h.shape[1]

    def k(agg_ref, h_ref, wroot_ref, br_ref, wnext_ref, hnew_ref, hrn_ref):
        agg = jnp.concatenate([agg_ref[0], agg_ref[1][:, :H - HW]], axis=1)
        root = lax.dot_general(
            h_ref[...], wroot_ref[...], (((1,), (1,)), ((), ())),
            preferred_element_type=jnp.float32)
        hnew = jnp.maximum(agg + root + br_ref[...], 0.0)
        hnew_ref[...] = hnew
        r = lax.dot_general(
            hnew, wnext_ref[...], (((1,), (1,)), ((), ())),
            preferred_element_type=jnp.float32)
        hrn_ref[0] = r[:, :HW]
        hrn_ref[1] = r[:, HW:]

    return pl.pallas_call(
        k,
        grid=(N // BLK,),
        in_specs=[pl.BlockSpec((NSC, BLK, HW), lambda i: (0, i, 0)),
                  pl.BlockSpec((BLK, d), lambda i: (i, 0)),
                  pl.BlockSpec((H, d), lambda i: (0, 0)),
                  pl.BlockSpec((1, H), lambda i: (0, 0)),
                  pl.BlockSpec((HP, H), lambda i: (0, 0))],
        out_specs=[pl.BlockSpec((BLK, H), lambda i: (i, 0)),
                   pl.BlockSpec((2, BLK, HW), lambda i: (0, i, 0))],
        out_shape=[jax.ShapeDtypeStruct((N, H), jnp.float32),
                   jax.ShapeDtypeStruct((2, N, HW), jnp.float32)],
    )(aggs, h, Wroot, br, Wnextp)


def _tc_final(aggs, h, Wroot, br, Wlin, blin):
    BLK = 1000
    d = h.shape[1]
    DO = Wlin.shape[0]

    def k(agg_ref, h_ref, wroot_ref, br_ref, wlin_ref, blin_ref, o_ref):
        agg = jnp.concatenate([agg_ref[0], agg_ref[1][:, :H - HW]], axis=1)
        root = lax.dot_general(
            h_ref[...], wroot_ref[...], (((1,), (1,)), ((), ())),
            preferred_element_type=jnp.float32)
        hnew = jnp.maximum(agg + root + br_ref[...], 0.0)
        o_ref[...] = lax.dot_general(
            hnew, wlin_ref[...], (((1,), (1,)), ((), ())),
            preferred_element_type=jnp.float32) + blin_ref[...]

    return pl.pallas_call(
        k,
        grid=(N // BLK,),
        in_specs=[pl.BlockSpec((NSC, BLK, HW), lambda i: (0, i, 0)),
                  pl.BlockSpec((BLK, d), lambda i: (i, 0)),
                  pl.BlockSpec((H, d), lambda i: (0, 0)),
                  pl.BlockSpec((1, H), lambda i: (0, 0)),
                  pl.BlockSpec((DO, H), lambda i: (0, 0)),
                  pl.BlockSpec((1, DO), lambda i: (0, 0))],
        out_specs=pl.BlockSpec((BLK, DO), lambda i: (i, 0)),
        out_shape=jax.ShapeDtypeStruct((N, DO), jnp.float32),
    )(aggs, h, Wroot, br, Wlin, blin)


def _pad_w(Wr):
    return jnp.pad(Wr, ((0, HP - H), (0, 0)))


def kernel(x, edge_index, W_rel0, b_rel0, W_root0, W_rel1, b_rel1, W_root1,
           W_rel2, b_rel2, W_root2, W_lin, b_lin):
    src = edge_index[0]
    dst = edge_index[1]
    pad = EPT_PAD * NSUB - E
    src0 = jnp.pad(src, (0, pad), constant_values=0
                   ).reshape(NSUB, EPT_PAD)
    srcp = jnp.stack([src0, src0 + N])
    dstp = jnp.pad(dst, (0, pad), constant_values=N
                   ).reshape(NSUB, NCHUNK, CHUNK)
    zeros = jnp.zeros((ROWS_PER_SUB, HW), jnp.float32)

    def agg(hr):
        return _sc_aggregate(hr.reshape(2 * N, HW), srcp, dstp, zeros,
                             col_split=False)

    agg0 = _sc_aggregate(x, src0, dstp, zeros, col_split=True)
    h1, hr1 = _tc_layer0(agg0, x, W_rel0, W_root0, b_rel0.reshape(1, -1),
                         _pad_w(W_rel1))
    agg1 = agg(hr1)
    h2, hr2 = _tc_layer(agg1, h1, W_root1, b_rel1.reshape(1, -1),
                        _pad_w(W_rel2))
    agg2 = agg(hr2)
    return _tc_final(agg2, h2, W_root2, b_rel2.reshape(1, -1),
                     W_lin, b_lin.reshape(1, -1))
